# baseline (speedup 1.0000x reference)
import functools

import jax
import jax.numpy as jnp
from jax import lax
from jax.experimental import pallas as pl
from jax.experimental.pallas import tpu as pltpu
from jax.experimental.pallas import tpu_sc as plsc

_N_NODES = 10000
_N_EDGES = 320000
_D_FEAT = 128
_D_EDGE = 16
_D_GLOBAL = 16
_N_BATCHES = 8
_IN_DIM = _D_FEAT + _D_EDGE + _D_GLOBAL
_LATENT = 32
_OUT_DIM = 128

_NC = 2
_NS = 16
_NCH = _N_EDGES // 128
_HROWS = _NCH // _NC
_RROWS = 125
_NROUND = _HROWS // _RROWS
_CROWS = _HROWS // _NS
_CEXTRA = _HROWS - _CROWS * _NS

_BLK = 2048
_GRID = -(-_N_NODES // _BLK)


def _sc_scatter_mean_partials(e4, ei3, zeros_n):
    mesh = plsc.VectorSubcoreMesh(core_axis_name="c", subcore_axis_name="s")

    @functools.partial(
        pl.kernel,
        out_type=(
            jax.ShapeDtypeStruct((_NC, _NS, _N_NODES), jnp.float32),
            jax.ShapeDtypeStruct((_NC, _NS, _N_NODES), jnp.float32),
        ),
        mesh=mesh,
        scratch_types=[
            pltpu.VMEM((_N_NODES,), jnp.float32),
            pltpu.VMEM((_N_NODES,), jnp.float32),
            pltpu.VMEM((_RROWS, 128), jnp.float32),
            pltpu.VMEM((_RROWS, 128), jnp.float32),
            pltpu.VMEM((_RROWS, 128), jnp.int32),
            pltpu.VMEM((_RROWS, 128), jnp.int32),
            pltpu.SemaphoreType.DMA,
            pltpu.SemaphoreType.DMA,
        ],
        compiler_params=pltpu.CompilerParams(use_tc_tiling_on_sc=False,
                                             needs_layout_passes=False),
    )
    def scatter_kernel(e4_hbm, ei3_hbm, zn_hbm, sums_hbm, cnts_hbm,
                       acc, cacc, strip0, strip1, recv0, recv1,
                       sem0, sem1):
        cid = lax.axis_index("c")
        sid = lax.axis_index("s")
        tr = sid // 8
        l0 = (sid % 8) * 128
        half0 = cid * _HROWS

        pltpu.sync_copy(zn_hbm, acc)
        pltpu.sync_copy(zn_hbm, cacc)

        strip_b = (strip0, strip1)
        recv_b = (recv0, recv1)
        sem = (sem0, sem1)
        ones16 = jnp.ones((16,), jnp.float32)

        def gather_round(rr, s):
            c0 = half0 + rr * _RROWS
            g1 = pltpu.async_copy(
                e4_hbm.at[tr, pl.ds(c0, _RROWS), pl.ds(l0, 128)],
                strip_b[s], sem[s])
            g2 = pltpu.async_copy(
                ei3_hbm.at[pl.ds(c0, _RROWS), pl.ds(128, 128)],
                recv_b[s], sem[s])
            return (g1, g2)

        def accum_round(s):
            sv = strip_b[s]
            rv = recv_b[s]

            @pl.loop(0, _RROWS, unroll=4)
            def _row(i):
                for g in range(8):
                    idxv = rv[i, pl.ds(16 * g, 16)]
                    valv = sv[i, pl.ds(16 * g, 16)]
                    plsc.addupdate_scatter(acc, [idxv], valv)

        @pl.loop(0, _NROUND, step=2)
        def _pair(rr):
            g0 = gather_round(rr, 0)
            g1 = gather_round(rr + 1, 1)
            for d in g0:
                d.wait()
            accum_round(0)
            for d in g1:
                d.wait()
            accum_round(1)

        crow0 = half0 + sid * _CROWS
        dc = pltpu.async_copy(ei3_hbm.at[pl.ds(crow0, _CROWS), pl.ds(128, 128)],
                              recv0.at[pl.ds(0, _CROWS), :], sem0)
        dc.wait()

        @pl.loop(0, _CROWS, unroll=4)
        def _crow(i):
            for g in range(8):
                idxv = recv0[i, pl.ds(16 * g, 16)]
                plsc.addupdate_scatter(cacc, [idxv], ones16)

        @pl.when(sid == _NS - 1)
        def _cextra():
            dx = pltpu.async_copy(
                ei3_hbm.at[pl.ds(half0 + _NS * _CROWS, _CEXTRA), pl.ds(128, 128)],
                recv1.at[pl.ds(0, _CEXTRA), :], sem1)
            dx.wait()

            @pl.loop(0, _CEXTRA)
            def _xrow(i):
                for g in range(8):
                    idxv = recv1[i, pl.ds(16 * g, 16)]
                    plsc.addupdate_scatter(cacc, [idxv], ones16)

        pltpu.sync_copy(acc, sums_hbm.at[cid, sid, :])
        pltpu.sync_copy(cacc, cnts_hbm.at[cid, sid, :])

    return scatter_kernel(e4, ei3, zeros_n)


def _tc_mlp_kernel(na_ref, s_ref, c_ref, g_ref, ng_ref,
                   W1_ref, b1_ref, W2_ref, b2_ref, out_ref):
    sT = s_ref[0] + s_ref[1]
    cnt = jnp.sum(c_ref[...], axis=(0, 1))
    aggT = sT / jnp.maximum(cnt, 1.0)[None, :]
    na = na_ref[...]
    ng = ng_ref[0]
    iota = lax.broadcasted_iota(jnp.int32, (_BLK, _N_BATCHES), 1)
    onehot = (ng[:, None] == iota).astype(jnp.float32)
    gW = jnp.dot(g_ref[...], W1_ref[_D_FEAT + _D_EDGE:, :],
                 preferred_element_type=jnp.float32)
    h_e = lax.dot_general(aggT, W1_ref[_D_FEAT:_D_FEAT + _D_EDGE, :],
                          ((( 0,), (0,)), ((), ())),
                          preferred_element_type=jnp.float32)
    h = (jnp.dot(na, W1_ref[:_D_FEAT, :], preferred_element_type=jnp.float32)
         + h_e
         + jnp.dot(onehot, gW, preferred_element_type=jnp.float32)
         + b1_ref[...])
    h = jnp.maximum(h, 0.0)
    out_ref[...] = jnp.dot(h, W2_ref[...],
                           preferred_element_type=jnp.float32) + b2_ref[...]


def _tc_mlp(node_attr, sums, cnts, global_attr, ng2, W1, b1r, W2, b2r):
    return pl.pallas_call(
        _tc_mlp_kernel,
        grid=(_GRID,),
        in_specs=[
            pl.BlockSpec((_BLK, _D_FEAT), lambda i: (i, 0)),
            pl.BlockSpec((_NC, _NS, _BLK), lambda i: (0, 0, i)),
            pl.BlockSpec((_NC, _NS, _BLK), lambda i: (0, 0, i)),
            pl.BlockSpec((_N_BATCHES, _D_GLOBAL), lambda i: (0, 0)),
            pl.BlockSpec((1, _BLK), lambda i: (0, i)),
            pl.BlockSpec((_IN_DIM, _LATENT), lambda i: (0, 0)),
            pl.BlockSpec((1, _LATENT), lambda i: (0, 0)),
            pl.BlockSpec((_LATENT, _OUT_DIM), lambda i: (0, 0)),
            pl.BlockSpec((1, _OUT_DIM), lambda i: (0, 0)),
        ],
        out_specs=pl.BlockSpec((_BLK, _OUT_DIM), lambda i: (i, 0)),
        out_shape=jax.ShapeDtypeStruct((_N_NODES, _OUT_DIM), jnp.float32),
    )(node_attr, sums, cnts, global_attr, ng2, W1, b1r, W2, b2r)


def kernel(node_attr, edge_attr, global_attr, edge_index, ng_index, eg_index,
           W1, b1, W2, b2):
    e4 = edge_attr.T.reshape(2, 8, _NCH, 128).transpose(0, 2, 1, 3)
    e4 = e4.reshape(2, _NCH, 1024)
    ei3 = edge_index.T.reshape(_NCH, 128, 2).transpose(0, 2, 1)
    ei3 = ei3.reshape(_NCH, 256)
    zeros_n = jnp.zeros((_N_NODES,), jnp.float32)
    sums, cnts = _sc_scatter_mean_partials(e4, ei3, zeros_n)
    ng2 = ng_index.reshape(1, _N_NODES)
    return _tc_mlp(node_attr, sums, cnts, global_attr, ng2,
                   W1, b1.reshape(1, -1), W2, b2.reshape(1, -1))


# speedup vs baseline: 2.3657x; 2.3657x over previous
import functools

import jax
import jax.numpy as jnp
from jax import lax
from jax.experimental import pallas as pl
from jax.experimental.pallas import tpu as pltpu
from jax.experimental.pallas import tpu_sc as plsc

_N_NODES = 10000
_N_EDGES = 320000
_D_FEAT = 128
_D_EDGE = 16
_D_GLOBAL = 16
_N_BATCHES = 8
_IN_DIM = _D_FEAT + _D_EDGE + _D_GLOBAL
_LATENT = 32
_OUT_DIM = 128

_NC = 2
_NS = 16
_NCH = _N_EDGES // 128
_HROWS = _NCH // _NC
_RROWS = 125
_NROUND = _HROWS // _RROWS
_CROWS = _HROWS // _NS
_CEXTRA = _HROWS - _CROWS * _NS

_BLK = 2048
_GRID = -(-_N_NODES // _BLK)


_HEDGES = _N_EDGES // _NC
_REDGES = 16000
_NROUND2 = _HEDGES // _REDGES
_TEDGES = _HEDGES // _NS


def _sc_scatter_mean_partials(eT, eidx, zeros_n):
    mesh = plsc.VectorSubcoreMesh(core_axis_name="c", subcore_axis_name="s")

    @functools.partial(
        pl.kernel,
        out_type=(
            jax.ShapeDtypeStruct((_NC, _NS, _N_NODES), jnp.float32),
            jax.ShapeDtypeStruct((_NC, _NS, _N_NODES), jnp.float32),
        ),
        mesh=mesh,
        scratch_types=[
            pltpu.VMEM((_N_NODES,), jnp.float32),
            pltpu.VMEM((_N_NODES,), jnp.float32),
            pltpu.VMEM((_N_NODES,), jnp.float32),
            pltpu.VMEM((_REDGES,), jnp.float32),
            pltpu.VMEM((_REDGES,), jnp.float32),
            pltpu.VMEM((_REDGES,), jnp.int32),
            pltpu.VMEM((_REDGES,), jnp.int32),
            pltpu.SemaphoreType.DMA,
            pltpu.SemaphoreType.DMA,
        ],
        compiler_params=pltpu.CompilerParams(use_tc_tiling_on_sc=False,
                                             needs_layout_passes=False),
    )
    def scatter_kernel(eT_hbm, eidx_hbm, zn_hbm, sums_hbm, cnts_hbm,
                       accA, accB, cacc, strip0, strip1, recv0, recv1,
                       sem0, sem1):
        cid = lax.axis_index("c")
        sid = lax.axis_index("s")
        e0 = cid * _HEDGES

        pltpu.sync_copy(zn_hbm, accA)
        pltpu.sync_copy(zn_hbm, accB)
        pltpu.sync_copy(zn_hbm, cacc)

        strip_b = (strip0, strip1)
        recv_b = (recv0, recv1)
        sem = (sem0, sem1)
        ones16 = jnp.ones((16,), jnp.float32)

        def gather_round(rr, s):
            off = e0 + rr * _REDGES
            g1 = pltpu.async_copy(eT_hbm.at[sid, pl.ds(off, _REDGES)],
                                  strip_b[s], sem[s])
            g2 = pltpu.async_copy(eidx_hbm.at[1, pl.ds(off, _REDGES)],
                                  recv_b[s], sem[s])
            return (g1, g2)

        def accum_round(s):
            sv = strip_b[s]
            rv = recv_b[s]

            @pl.loop(0, _REDGES // 32)
            def _pairg(i):
                idxa = rv[pl.ds(32 * i, 16)]
                vala = sv[pl.ds(32 * i, 16)]
                plsc.addupdate_scatter(accA, [idxa], vala)
                idxb = rv[pl.ds(32 * i + 16, 16)]
                valb = sv[pl.ds(32 * i + 16, 16)]
                plsc.addupdate_scatter(accB, [idxb], valb)

        @pl.loop(0, _NROUND2, step=2)
        def _pair(rr):
            g0 = gather_round(rr, 0)
            g1 = gather_round(rr + 1, 1)
            for d in g0:
                d.wait()
            accum_round(0)
            for d in g1:
                d.wait()
            accum_round(1)

        dc = pltpu.async_copy(
            eidx_hbm.at[1, pl.ds(e0 + sid * _TEDGES, _TEDGES)],
            recv0.at[pl.ds(0, _TEDGES)], sem0)
        dc.wait()

        @pl.loop(0, _TEDGES // 16)
        def _cg(i):
            idxv = recv0[pl.ds(16 * i, 16)]
            plsc.addupdate_scatter(cacc, [idxv], ones16)

        @pl.loop(0, _N_NODES // 16)
        def _merge(i):
            sl = pl.ds(16 * i, 16)
            accA[sl] = accA[sl] + accB[sl]

        pltpu.sync_copy(accA, sums_hbm.at[cid, sid, :])
        pltpu.sync_copy(cacc, cnts_hbm.at[cid, sid, :])

    return scatter_kernel(eT, eidx, zeros_n)


def _tc_mlp_kernel(na_ref, s_ref, c_ref, g_ref, ng_ref,
                   W1_ref, b1_ref, W2_ref, b2_ref, out_ref):
    sT = s_ref[0] + s_ref[1]
    cnt = jnp.sum(c_ref[...], axis=(0, 1))
    aggT = sT / jnp.maximum(cnt, 1.0)[None, :]
    na = na_ref[...]
    ng = ng_ref[0]
    iota = lax.broadcasted_iota(jnp.int32, (_BLK, _N_BATCHES), 1)
    onehot = (ng[:, None] == iota).astype(jnp.float32)
    gW = jnp.dot(g_ref[...], W1_ref[_D_FEAT + _D_EDGE:, :],
                 preferred_element_type=jnp.float32)
    h_e = lax.dot_general(aggT, W1_ref[_D_FEAT:_D_FEAT + _D_EDGE, :],
                          ((( 0,), (0,)), ((), ())),
                          preferred_element_type=jnp.float32)
    h = (jnp.dot(na, W1_ref[:_D_FEAT, :], preferred_element_type=jnp.float32)
         + h_e
         + jnp.dot(onehot, gW, preferred_element_type=jnp.float32)
         + b1_ref[...])
    h = jnp.maximum(h, 0.0)
    out_ref[...] = jnp.dot(h, W2_ref[...],
                           preferred_element_type=jnp.float32) + b2_ref[...]


def _tc_mlp(node_attr, sums, cnts, global_attr, ng2, W1, b1r, W2, b2r):
    return pl.pallas_call(
        _tc_mlp_kernel,
        grid=(_GRID,),
        in_specs=[
            pl.BlockSpec((_BLK, _D_FEAT), lambda i: (i, 0)),
            pl.BlockSpec((_NC, _NS, _BLK), lambda i: (0, 0, i)),
            pl.BlockSpec((_NC, _NS, _BLK), lambda i: (0, 0, i)),
            pl.BlockSpec((_N_BATCHES, _D_GLOBAL), lambda i: (0, 0)),
            pl.BlockSpec((1, _BLK), lambda i: (0, i)),
            pl.BlockSpec((_IN_DIM, _LATENT), lambda i: (0, 0)),
            pl.BlockSpec((1, _LATENT), lambda i: (0, 0)),
            pl.BlockSpec((_LATENT, _OUT_DIM), lambda i: (0, 0)),
            pl.BlockSpec((1, _OUT_DIM), lambda i: (0, 0)),
        ],
        out_specs=pl.BlockSpec((_BLK, _OUT_DIM), lambda i: (i, 0)),
        out_shape=jax.ShapeDtypeStruct((_N_NODES, _OUT_DIM), jnp.float32),
    )(node_attr, sums, cnts, global_attr, ng2, W1, b1r, W2, b2r)


def kernel(node_attr, edge_attr, global_attr, edge_index, ng_index, eg_index,
           W1, b1, W2, b2):
    eT = edge_attr.T
    zeros_n = jnp.zeros((_N_NODES,), jnp.float32)
    sums, cnts = _sc_scatter_mean_partials(eT, edge_index, zeros_n)
    ng2 = ng_index.reshape(1, _N_NODES)
    return _tc_mlp(node_attr, sums, cnts, global_attr, ng2,
                   W1, b1.reshape(1, -1), W2, b2.reshape(1, -1))


# speedup vs baseline: 2.9720x; 1.2563x over previous
import functools

import jax
import jax.numpy as jnp
from jax import lax
from jax.experimental import pallas as pl
from jax.experimental.pallas import tpu as pltpu
from jax.experimental.pallas import tpu_sc as plsc

_N_NODES = 10000
_N_EDGES = 320000
_D_FEAT = 128
_D_EDGE = 16
_D_GLOBAL = 16
_N_BATCHES = 8
_IN_DIM = _D_FEAT + _D_EDGE + _D_GLOBAL
_LATENT = 32
_OUT_DIM = 128

_NC = 2
_NS = 16
_NCH = _N_EDGES // 128
_HROWS = _NCH // _NC
_RROWS = 125
_NROUND = _HROWS // _RROWS
_CROWS = _HROWS // _NS
_CEXTRA = _HROWS - _CROWS * _NS

_BLK = 2048
_GRID = -(-_N_NODES // _BLK)


_HEDGES = _N_EDGES // _NC
_REDGES = 16000
_NROUND2 = _HEDGES // _REDGES
_TEDGES = _HEDGES // _NS


def _sc_scatter_mean_partials(eT, eidx, zeros_n):
    mesh = plsc.VectorSubcoreMesh(core_axis_name="c", subcore_axis_name="s")

    @functools.partial(
        pl.kernel,
        out_type=(
            jax.ShapeDtypeStruct((_NC, _NS, _N_NODES), jnp.float32),
            jax.ShapeDtypeStruct((_NC, _NS, _N_NODES), jnp.float32),
        ),
        mesh=mesh,
        scratch_types=[
            pltpu.VMEM((_N_NODES,), jnp.float32),
            pltpu.VMEM((_N_NODES,), jnp.float32),
            pltpu.VMEM((_N_NODES,), jnp.float32),
            pltpu.VMEM((_REDGES,), jnp.float32),
            pltpu.VMEM((_REDGES,), jnp.float32),
            pltpu.VMEM((_REDGES,), jnp.int32),
            pltpu.VMEM((_REDGES,), jnp.int32),
            pltpu.SemaphoreType.DMA,
            pltpu.SemaphoreType.DMA,
        ],
        compiler_params=pltpu.CompilerParams(use_tc_tiling_on_sc=False,
                                             needs_layout_passes=False),
    )
    def scatter_kernel(eT_hbm, eidx_hbm, zn_hbm, sums_hbm, cnts_hbm,
                       accA, accB, cacc, strip0, strip1, recv0, recv1,
                       sem0, sem1):
        cid = lax.axis_index("c")
        sid = lax.axis_index("s")
        e0 = cid * _HEDGES

        pltpu.sync_copy(zn_hbm, accA)
        pltpu.sync_copy(zn_hbm, accB)
        pltpu.sync_copy(zn_hbm, cacc)

        strip_b = (strip0, strip1)
        recv_b = (recv0, recv1)
        sem = (sem0, sem1)
        ones16 = jnp.ones((16,), jnp.float32)

        def gather_round(rr, s):
            off = e0 + rr * _REDGES
            g1 = pltpu.async_copy(eT_hbm.at[sid, pl.ds(off, _REDGES)],
                                  strip_b[s], sem[s])
            g2 = pltpu.async_copy(eidx_hbm.at[1, pl.ds(off, _REDGES)],
                                  recv_b[s], sem[s])
            return (g1, g2)

        def accum_round(s):
            sv = strip_b[s]
            rv = recv_b[s]

            @plsc.parallel_loop(0, _REDGES // 32)
            def _pairg(i):
                idxa = rv[pl.ds(32 * i, 16)]
                vala = sv[pl.ds(32 * i, 16)]
                plsc.addupdate_scatter(accA, [idxa], vala)
                idxb = rv[pl.ds(32 * i + 16, 16)]
                valb = sv[pl.ds(32 * i + 16, 16)]
                plsc.addupdate_scatter(accB, [idxb], valb)

        @pl.loop(0, _NROUND2, step=2)
        def _pair(rr):
            g0 = gather_round(rr, 0)
            g1 = gather_round(rr + 1, 1)
            for d in g0:
                d.wait()
            accum_round(0)
            for d in g1:
                d.wait()
            accum_round(1)

        dc = pltpu.async_copy(
            eidx_hbm.at[1, pl.ds(e0 + sid * _TEDGES, _TEDGES)],
            recv0.at[pl.ds(0, _TEDGES)], sem0)
        dc.wait()

        @plsc.parallel_loop(0, _TEDGES // 16)
        def _cg(i):
            idxv = recv0[pl.ds(16 * i, 16)]
            plsc.addupdate_scatter(cacc, [idxv], ones16)

        @pl.loop(0, _N_NODES // 16)
        def _merge(i):
            sl = pl.ds(16 * i, 16)
            accA[sl] = accA[sl] + accB[sl]

        pltpu.sync_copy(accA, sums_hbm.at[cid, sid, :])
        pltpu.sync_copy(cacc, cnts_hbm.at[cid, sid, :])

    return scatter_kernel(eT, eidx, zeros_n)


def _tc_mlp_kernel(na_ref, s_ref, c_ref, g_ref, ng_ref,
                   W1_ref, b1_ref, W2_ref, b2_ref, out_ref):
    sT = s_ref[0] + s_ref[1]
    cnt = jnp.sum(c_ref[...], axis=(0, 1))
    aggT = sT / jnp.maximum(cnt, 1.0)[None, :]
    na = na_ref[...]
    ng = ng_ref[0]
    iota = lax.broadcasted_iota(jnp.int32, (_BLK, _N_BATCHES), 1)
    onehot = (ng[:, None] == iota).astype(jnp.float32)
    gW = jnp.dot(g_ref[...], W1_ref[_D_FEAT + _D_EDGE:, :],
                 preferred_element_type=jnp.float32)
    h_e = lax.dot_general(aggT, W1_ref[_D_FEAT:_D_FEAT + _D_EDGE, :],
                          ((( 0,), (0,)), ((), ())),
                          preferred_element_type=jnp.float32)
    h = (jnp.dot(na, W1_ref[:_D_FEAT, :], preferred_element_type=jnp.float32)
         + h_e
         + jnp.dot(onehot, gW, preferred_element_type=jnp.float32)
         + b1_ref[...])
    h = jnp.maximum(h, 0.0)
    out_ref[...] = jnp.dot(h, W2_ref[...],
                           preferred_element_type=jnp.float32) + b2_ref[...]


def _tc_mlp(node_attr, sums, cnts, global_attr, ng2, W1, b1r, W2, b2r):
    return pl.pallas_call(
        _tc_mlp_kernel,
        grid=(_GRID,),
        in_specs=[
            pl.BlockSpec((_BLK, _D_FEAT), lambda i: (i, 0)),
            pl.BlockSpec((_NC, _NS, _BLK), lambda i: (0, 0, i)),
            pl.BlockSpec((_NC, _NS, _BLK), lambda i: (0, 0, i)),
            pl.BlockSpec((_N_BATCHES, _D_GLOBAL), lambda i: (0, 0)),
            pl.BlockSpec((1, _BLK), lambda i: (0, i)),
            pl.BlockSpec((_IN_DIM, _LATENT), lambda i: (0, 0)),
            pl.BlockSpec((1, _LATENT), lambda i: (0, 0)),
            pl.BlockSpec((_LATENT, _OUT_DIM), lambda i: (0, 0)),
            pl.BlockSpec((1, _OUT_DIM), lambda i: (0, 0)),
        ],
        out_specs=pl.BlockSpec((_BLK, _OUT_DIM), lambda i: (i, 0)),
        out_shape=jax.ShapeDtypeStruct((_N_NODES, _OUT_DIM), jnp.float32),
    )(node_attr, sums, cnts, global_attr, ng2, W1, b1r, W2, b2r)


def kernel(node_attr, edge_attr, global_attr, edge_index, ng_index, eg_index,
           W1, b1, W2, b2):
    eT = edge_attr.T
    zeros_n = jnp.zeros((_N_NODES,), jnp.float32)
    sums, cnts = _sc_scatter_mean_partials(eT, edge_index, zeros_n)
    ng2 = ng_index.reshape(1, _N_NODES)
    return _tc_mlp(node_attr, sums, cnts, global_attr, ng2,
                   W1, b1.reshape(1, -1), W2, b2.reshape(1, -1))


# speedup vs baseline: 3.1322x; 1.0539x over previous
import functools

import jax
import jax.numpy as jnp
from jax import lax
from jax.experimental import pallas as pl
from jax.experimental.pallas import tpu as pltpu
from jax.experimental.pallas import tpu_sc as plsc

_N_NODES = 10000
_N_EDGES = 320000
_D_FEAT = 128
_D_EDGE = 16
_D_GLOBAL = 16
_N_BATCHES = 8
_IN_DIM = _D_FEAT + _D_EDGE + _D_GLOBAL
_LATENT = 32
_OUT_DIM = 128

_NC = 2
_NS = 16
_NCH = _N_EDGES // 128
_HROWS = _NCH // _NC
_RROWS = 125
_NROUND = _HROWS // _RROWS
_CROWS = _HROWS // _NS
_CEXTRA = _HROWS - _CROWS * _NS

_BLK = 2048
_GRID = -(-_N_NODES // _BLK)


_HEDGES = _N_EDGES // _NC
_REDGES = 16000
_NROUND2 = _HEDGES // _REDGES
_TEDGES = _HEDGES // _NS


def _sc_scatter_mean_partials(eT, eidx, zeros_n):
    mesh = plsc.VectorSubcoreMesh(core_axis_name="c", subcore_axis_name="s")

    @functools.partial(
        pl.kernel,
        out_type=(
            jax.ShapeDtypeStruct((_NC, _NS, _N_NODES), jnp.float32),
            jax.ShapeDtypeStruct((_NC, _NS, _N_NODES), jnp.float32),
        ),
        mesh=mesh,
        scratch_types=[
            pltpu.VMEM((_N_NODES,), jnp.float32),
            pltpu.VMEM((_N_NODES,), jnp.float32),
            pltpu.VMEM((_N_NODES,), jnp.float32),
            pltpu.VMEM((_REDGES,), jnp.float32),
            pltpu.VMEM((_REDGES,), jnp.float32),
            pltpu.VMEM((_REDGES,), jnp.int32),
            pltpu.VMEM((_REDGES,), jnp.int32),
            pltpu.SemaphoreType.DMA,
            pltpu.SemaphoreType.DMA,
        ],
        compiler_params=pltpu.CompilerParams(use_tc_tiling_on_sc=False,
                                             needs_layout_passes=False),
    )
    def scatter_kernel(eT_hbm, eidx_hbm, zn_hbm, sums_hbm, cnts_hbm,
                       accA, accB, cacc, strip0, strip1, recv0, recv1,
                       sem0, sem1):
        cid = lax.axis_index("c")
        sid = lax.axis_index("s")
        e0 = cid * _HEDGES

        pltpu.sync_copy(zn_hbm, accA)
        pltpu.sync_copy(zn_hbm, accB)
        pltpu.sync_copy(zn_hbm, cacc)

        strip_b = (strip0, strip1)
        recv_b = (recv0, recv1)
        sem = (sem0, sem1)
        ones16 = jnp.ones((16,), jnp.float32)

        def gather_round(rr, s):
            off = e0 + rr * _REDGES
            g1 = pltpu.async_copy(eT_hbm.at[sid, pl.ds(off, _REDGES)],
                                  strip_b[s], sem[s])
            g2 = pltpu.async_copy(eidx_hbm.at[1, pl.ds(off, _REDGES)],
                                  recv_b[s], sem[s])
            return (g1, g2)

        def accum_round(s):
            sv = strip_b[s]
            rv = recv_b[s]

            @plsc.parallel_loop(0, _REDGES // 32, unroll=2)
            def _pairg(i):
                idxa = rv[pl.ds(32 * i, 16)]
                vala = sv[pl.ds(32 * i, 16)]
                plsc.addupdate_scatter(accA, [idxa], vala)
                idxb = rv[pl.ds(32 * i + 16, 16)]
                valb = sv[pl.ds(32 * i + 16, 16)]
                plsc.addupdate_scatter(accB, [idxb], valb)

        @pl.loop(0, _NROUND2, step=2)
        def _pair(rr):
            g0 = gather_round(rr, 0)
            g1 = gather_round(rr + 1, 1)
            for d in g0:
                d.wait()
            accum_round(0)
            for d in g1:
                d.wait()
            accum_round(1)

        dc = pltpu.async_copy(
            eidx_hbm.at[1, pl.ds(e0 + sid * _TEDGES, _TEDGES)],
            recv0.at[pl.ds(0, _TEDGES)], sem0)
        dc.wait()

        @plsc.parallel_loop(0, _TEDGES // 16)
        def _cg(i):
            idxv = recv0[pl.ds(16 * i, 16)]
            plsc.addupdate_scatter(cacc, [idxv], ones16)

        @plsc.parallel_loop(0, _N_NODES // 16)
        def _merge(i):
            sl = pl.ds(16 * i, 16)
            accA[sl] = accA[sl] + accB[sl]

        pltpu.sync_copy(accA, sums_hbm.at[cid, sid, :])
        pltpu.sync_copy(cacc, cnts_hbm.at[cid, sid, :])

    return scatter_kernel(eT, eidx, zeros_n)


def _tc_mlp_kernel(na_ref, s_ref, c_ref, g_ref, ng_ref,
                   W1_ref, b1_ref, W2_ref, b2_ref, out_ref):
    sT = s_ref[0] + s_ref[1]
    cnt = jnp.sum(c_ref[...], axis=(0, 1))
    aggT = sT / jnp.maximum(cnt, 1.0)[None, :]
    na = na_ref[...]
    ng = ng_ref[0]
    iota = lax.broadcasted_iota(jnp.int32, (_BLK, _N_BATCHES), 1)
    onehot = (ng[:, None] == iota).astype(jnp.float32)
    gW = jnp.dot(g_ref[...], W1_ref[_D_FEAT + _D_EDGE:, :],
                 preferred_element_type=jnp.float32)
    h_e = lax.dot_general(aggT, W1_ref[_D_FEAT:_D_FEAT + _D_EDGE, :],
                          ((( 0,), (0,)), ((), ())),
                          preferred_element_type=jnp.float32)
    h = (jnp.dot(na, W1_ref[:_D_FEAT, :], preferred_element_type=jnp.float32)
         + h_e
         + jnp.dot(onehot, gW, preferred_element_type=jnp.float32)
         + b1_ref[...])
    h = jnp.maximum(h, 0.0)
    out_ref[...] = jnp.dot(h, W2_ref[...],
                           preferred_element_type=jnp.float32) + b2_ref[...]


def _tc_mlp(node_attr, sums, cnts, global_attr, ng2, W1, b1r, W2, b2r):
    return pl.pallas_call(
        _tc_mlp_kernel,
        grid=(_GRID,),
        in_specs=[
            pl.BlockSpec((_BLK, _D_FEAT), lambda i: (i, 0)),
            pl.BlockSpec((_NC, _NS, _BLK), lambda i: (0, 0, i)),
            pl.BlockSpec((_NC, _NS, _BLK), lambda i: (0, 0, i)),
            pl.BlockSpec((_N_BATCHES, _D_GLOBAL), lambda i: (0, 0)),
            pl.BlockSpec((1, _BLK), lambda i: (0, i)),
            pl.BlockSpec((_IN_DIM, _LATENT), lambda i: (0, 0)),
            pl.BlockSpec((1, _LATENT), lambda i: (0, 0)),
            pl.BlockSpec((_LATENT, _OUT_DIM), lambda i: (0, 0)),
            pl.BlockSpec((1, _OUT_DIM), lambda i: (0, 0)),
        ],
        out_specs=pl.BlockSpec((_BLK, _OUT_DIM), lambda i: (i, 0)),
        out_shape=jax.ShapeDtypeStruct((_N_NODES, _OUT_DIM), jnp.float32),
    )(node_attr, sums, cnts, global_attr, ng2, W1, b1r, W2, b2r)


def kernel(node_attr, edge_attr, global_attr, edge_index, ng_index, eg_index,
           W1, b1, W2, b2):
    eT = edge_attr.T
    zeros_n = jnp.zeros((_N_NODES,), jnp.float32)
    sums, cnts = _sc_scatter_mean_partials(eT, edge_index, zeros_n)
    ng2 = ng_index.reshape(1, _N_NODES)
    return _tc_mlp(node_attr, sums, cnts, global_attr, ng2,
                   W1, b1.reshape(1, -1), W2, b2.reshape(1, -1))


# speedup vs baseline: 3.1408x; 1.0028x over previous
import functools

import jax
import jax.numpy as jnp
from jax import lax
from jax.experimental import pallas as pl
from jax.experimental.pallas import tpu as pltpu
from jax.experimental.pallas import tpu_sc as plsc

_N_NODES = 10000
_N_EDGES = 320000
_D_FEAT = 128
_D_EDGE = 16
_D_GLOBAL = 16
_N_BATCHES = 8
_IN_DIM = _D_FEAT + _D_EDGE + _D_GLOBAL
_LATENT = 32
_OUT_DIM = 128

_NC = 2
_NS = 16
_NCH = _N_EDGES // 128
_HROWS = _NCH // _NC
_RROWS = 125
_NROUND = _HROWS // _RROWS
_CROWS = _HROWS // _NS
_CEXTRA = _HROWS - _CROWS * _NS

_BLK = 2048
_GRID = -(-_N_NODES // _BLK)


_HEDGES = _N_EDGES // _NC
_REDGES = 16000
_NROUND2 = _HEDGES // _REDGES
_TEDGES = _HEDGES // _NS


def _sc_scatter_mean_partials(eT, eidx, zeros_n):
    mesh = plsc.VectorSubcoreMesh(core_axis_name="c", subcore_axis_name="s")

    @functools.partial(
        pl.kernel,
        out_type=(
            jax.ShapeDtypeStruct((_NC, _NS, _N_NODES), jnp.float32),
            jax.ShapeDtypeStruct((_NC, _NS, _N_NODES), jnp.float32),
        ),
        mesh=mesh,
        scratch_types=[
            pltpu.VMEM((_N_NODES,), jnp.float32),
            pltpu.VMEM((_N_NODES,), jnp.float32),
            pltpu.VMEM((_N_NODES,), jnp.float32),
            pltpu.VMEM((_REDGES,), jnp.float32),
            pltpu.VMEM((_REDGES,), jnp.float32),
            pltpu.VMEM((_REDGES,), jnp.int32),
            pltpu.VMEM((_REDGES,), jnp.int32),
            pltpu.SemaphoreType.DMA,
            pltpu.SemaphoreType.DMA,
        ],
        compiler_params=pltpu.CompilerParams(use_tc_tiling_on_sc=False,
                                             needs_layout_passes=False),
    )
    def scatter_kernel(eT_hbm, eidx_hbm, zn_hbm, sums_hbm, cnts_hbm,
                       accA, accB, cacc, strip0, strip1, recv0, recv1,
                       sem0, sem1):
        cid = lax.axis_index("c")
        sid = lax.axis_index("s")
        e0 = cid * _HEDGES

        pltpu.sync_copy(zn_hbm, accA)
        pltpu.sync_copy(zn_hbm, accB)
        pltpu.sync_copy(zn_hbm, cacc)

        strip_b = (strip0, strip1)
        recv_b = (recv0, recv1)
        sem = (sem0, sem1)
        ones16 = jnp.ones((16,), jnp.float32)

        def gather_round(rr, s):
            off = e0 + rr * _REDGES
            g1 = pltpu.async_copy(eT_hbm.at[sid, pl.ds(off, _REDGES)],
                                  strip_b[s], sem[s])
            g2 = pltpu.async_copy(eidx_hbm.at[1, pl.ds(off, _REDGES)],
                                  recv_b[s], sem[s])
            return (g1, g2)

        def accum_round(s):
            sv = strip_b[s]
            rv = recv_b[s]

            @plsc.parallel_loop(0, _REDGES // 32, unroll=4)
            def _pairg(i):
                idxa = rv[pl.ds(32 * i, 16)]
                vala = sv[pl.ds(32 * i, 16)]
                plsc.addupdate_scatter(accA, [idxa], vala)
                idxb = rv[pl.ds(32 * i + 16, 16)]
                valb = sv[pl.ds(32 * i + 16, 16)]
                plsc.addupdate_scatter(accB, [idxb], valb)

        @pl.loop(0, _NROUND2, step=2)
        def _pair(rr):
            g0 = gather_round(rr, 0)
            g1 = gather_round(rr + 1, 1)
            for d in g0:
                d.wait()
            accum_round(0)
            for d in g1:
                d.wait()
            accum_round(1)

        dc = pltpu.async_copy(
            eidx_hbm.at[1, pl.ds(e0 + sid * _TEDGES, _TEDGES)],
            recv0.at[pl.ds(0, _TEDGES)], sem0)
        dc.wait()

        @plsc.parallel_loop(0, _TEDGES // 16)
        def _cg(i):
            idxv = recv0[pl.ds(16 * i, 16)]
            plsc.addupdate_scatter(cacc, [idxv], ones16)

        @plsc.parallel_loop(0, _N_NODES // 16)
        def _merge(i):
            sl = pl.ds(16 * i, 16)
            accA[sl] = accA[sl] + accB[sl]

        pltpu.sync_copy(accA, sums_hbm.at[cid, sid, :])
        pltpu.sync_copy(cacc, cnts_hbm.at[cid, sid, :])

    return scatter_kernel(eT, eidx, zeros_n)


def _tc_mlp_kernel(na_ref, s_ref, c_ref, g_ref, ng_ref,
                   W1_ref, b1_ref, W2_ref, b2_ref, out_ref):
    sT = s_ref[0] + s_ref[1]
    cnt = jnp.sum(c_ref[...], axis=(0, 1))
    aggT = sT / jnp.maximum(cnt, 1.0)[None, :]
    na = na_ref[...]
    ng = ng_ref[0]
    iota = lax.broadcasted_iota(jnp.int32, (_BLK, _N_BATCHES), 1)
    onehot = (ng[:, None] == iota).astype(jnp.float32)
    gW = jnp.dot(g_ref[...], W1_ref[_D_FEAT + _D_EDGE:, :],
                 preferred_element_type=jnp.float32)
    h_e = lax.dot_general(aggT, W1_ref[_D_FEAT:_D_FEAT + _D_EDGE, :],
                          ((( 0,), (0,)), ((), ())),
                          preferred_element_type=jnp.float32)
    h = (jnp.dot(na, W1_ref[:_D_FEAT, :], preferred_element_type=jnp.float32)
         + h_e
         + jnp.dot(onehot, gW, preferred_element_type=jnp.float32)
         + b1_ref[...])
    h = jnp.maximum(h, 0.0)
    out_ref[...] = jnp.dot(h, W2_ref[...],
                           preferred_element_type=jnp.float32) + b2_ref[...]


def _tc_mlp(node_attr, sums, cnts, global_attr, ng2, W1, b1r, W2, b2r):
    return pl.pallas_call(
        _tc_mlp_kernel,
        grid=(_GRID,),
        in_specs=[
            pl.BlockSpec((_BLK, _D_FEAT), lambda i: (i, 0)),
            pl.BlockSpec((_NC, _NS, _BLK), lambda i: (0, 0, i)),
            pl.BlockSpec((_NC, _NS, _BLK), lambda i: (0, 0, i)),
            pl.BlockSpec((_N_BATCHES, _D_GLOBAL), lambda i: (0, 0)),
            pl.BlockSpec((1, _BLK), lambda i: (0, i)),
            pl.BlockSpec((_IN_DIM, _LATENT), lambda i: (0, 0)),
            pl.BlockSpec((1, _LATENT), lambda i: (0, 0)),
            pl.BlockSpec((_LATENT, _OUT_DIM), lambda i: (0, 0)),
            pl.BlockSpec((1, _OUT_DIM), lambda i: (0, 0)),
        ],
        out_specs=pl.BlockSpec((_BLK, _OUT_DIM), lambda i: (i, 0)),
        out_shape=jax.ShapeDtypeStruct((_N_NODES, _OUT_DIM), jnp.float32),
    )(node_attr, sums, cnts, global_attr, ng2, W1, b1r, W2, b2r)


def kernel(node_attr, edge_attr, global_attr, edge_index, ng_index, eg_index,
           W1, b1, W2, b2):
    eT = edge_attr.T
    zeros_n = jnp.zeros((_N_NODES,), jnp.float32)
    sums, cnts = _sc_scatter_mean_partials(eT, edge_index, zeros_n)
    ng2 = ng_index.reshape(1, _N_NODES)
    return _tc_mlp(node_attr, sums, cnts, global_attr, ng2,
                   W1, b1.reshape(1, -1), W2, b2.reshape(1, -1))


# speedup vs baseline: 3.1426x; 1.0006x over previous
import functools

import jax
import jax.numpy as jnp
from jax import lax
from jax.experimental import pallas as pl
from jax.experimental.pallas import tpu as pltpu
from jax.experimental.pallas import tpu_sc as plsc

_N_NODES = 10000
_N_EDGES = 320000
_D_FEAT = 128
_D_EDGE = 16
_D_GLOBAL = 16
_N_BATCHES = 8
_IN_DIM = _D_FEAT + _D_EDGE + _D_GLOBAL
_LATENT = 32
_OUT_DIM = 128

_NC = 2
_NS = 16

_BLK = 2048
_GRID = -(-_N_NODES // _BLK)


_HEDGES = _N_EDGES // _NC
_REDGES = 16000
_NROUND2 = _HEDGES // _REDGES
_TEDGES = _HEDGES // _NS


def _sc_scatter_mean_partials(eT, eidx, zeros_n):
    mesh = plsc.VectorSubcoreMesh(core_axis_name="c", subcore_axis_name="s")

    @functools.partial(
        pl.kernel,
        out_type=(
            jax.ShapeDtypeStruct((_NC, _NS, _N_NODES), jnp.float32),
            jax.ShapeDtypeStruct((_NC, _NS, _N_NODES), jnp.float32),
        ),
        mesh=mesh,
        scratch_types=[
            pltpu.VMEM((_N_NODES,), jnp.float32),
            pltpu.VMEM((_N_NODES,), jnp.float32),
            pltpu.VMEM((_N_NODES,), jnp.float32),
            pltpu.VMEM((_REDGES,), jnp.float32),
            pltpu.VMEM((_REDGES,), jnp.float32),
            pltpu.VMEM((_REDGES,), jnp.int32),
            pltpu.VMEM((_REDGES,), jnp.int32),
            pltpu.SemaphoreType.DMA,
            pltpu.SemaphoreType.DMA,
        ],
        compiler_params=pltpu.CompilerParams(use_tc_tiling_on_sc=False,
                                             needs_layout_passes=False),
    )
    def scatter_kernel(eT_hbm, eidx_hbm, zn_hbm, sums_hbm, cnts_hbm,
                       accA, accB, cacc, strip0, strip1, recv0, recv1,
                       sem0, sem1):
        cid = lax.axis_index("c")
        sid = lax.axis_index("s")
        e0 = cid * _HEDGES

        pltpu.sync_copy(zn_hbm, accA)
        pltpu.sync_copy(zn_hbm, accB)
        pltpu.sync_copy(zn_hbm, cacc)

        strip_b = (strip0, strip1)
        recv_b = (recv0, recv1)
        sem = (sem0, sem1)
        ones16 = jnp.ones((16,), jnp.float32)

        def gather_round(rr, s):
            off = e0 + rr * _REDGES
            g1 = pltpu.async_copy(eT_hbm.at[sid, pl.ds(off, _REDGES)],
                                  strip_b[s], sem[s])
            g2 = pltpu.async_copy(eidx_hbm.at[1, pl.ds(off, _REDGES)],
                                  recv_b[s], sem[s])
            return (g1, g2)

        def accum_round(s):
            sv = strip_b[s]
            rv = recv_b[s]

            @plsc.parallel_loop(0, _REDGES // 32, unroll=4)
            def _pairg(i):
                idxa = rv[pl.ds(32 * i, 16)]
                vala = sv[pl.ds(32 * i, 16)]
                plsc.addupdate_scatter(accA, [idxa], vala)
                idxb = rv[pl.ds(32 * i + 16, 16)]
                valb = sv[pl.ds(32 * i + 16, 16)]
                plsc.addupdate_scatter(accB, [idxb], valb)

        @pl.loop(0, _NROUND2, step=2)
        def _pair(rr):
            g0 = gather_round(rr, 0)
            g1 = gather_round(rr + 1, 1)
            for d in g0:
                d.wait()
            accum_round(0)
            for d in g1:
                d.wait()
            accum_round(1)

        dc = pltpu.async_copy(
            eidx_hbm.at[1, pl.ds(e0 + sid * _TEDGES, _TEDGES)],
            recv0.at[pl.ds(0, _TEDGES)], sem0)
        dc.wait()

        @plsc.parallel_loop(0, _TEDGES // 16)
        def _cg(i):
            idxv = recv0[pl.ds(16 * i, 16)]
            plsc.addupdate_scatter(cacc, [idxv], ones16)

        @plsc.parallel_loop(0, _N_NODES // 16)
        def _merge(i):
            sl = pl.ds(16 * i, 16)
            accA[sl] = accA[sl] + accB[sl]

        pltpu.sync_copy(accA, sums_hbm.at[cid, sid, :])
        pltpu.sync_copy(cacc, cnts_hbm.at[cid, sid, :])

    return scatter_kernel(eT, eidx, zeros_n)


def _tc_mlp_kernel(na_ref, s_ref, c_ref, g_ref, ng_ref,
                   W1_ref, b1_ref, W2_ref, b2_ref, out_ref):
    sT = s_ref[0] + s_ref[1]
    cnt = jnp.sum(c_ref[...], axis=(0, 1))
    aggT = sT / jnp.maximum(cnt, 1.0)[None, :]
    na = na_ref[...]
    ng = ng_ref[0]
    iota = lax.broadcasted_iota(jnp.int32, (_BLK, _N_BATCHES), 1)
    onehot = (ng[:, None] == iota).astype(jnp.float32)
    gW = jnp.dot(g_ref[...], W1_ref[_D_FEAT + _D_EDGE:, :],
                 preferred_element_type=jnp.float32)
    h_e = lax.dot_general(aggT, W1_ref[_D_FEAT:_D_FEAT + _D_EDGE, :],
                          ((( 0,), (0,)), ((), ())),
                          preferred_element_type=jnp.float32)
    h = (jnp.dot(na, W1_ref[:_D_FEAT, :], preferred_element_type=jnp.float32)
         + h_e
         + jnp.dot(onehot, gW, preferred_element_type=jnp.float32)
         + b1_ref[...])
    h = jnp.maximum(h, 0.0)
    out_ref[...] = jnp.dot(h, W2_ref[...],
                           preferred_element_type=jnp.float32) + b2_ref[...]


def _tc_mlp(node_attr, sums, cnts, global_attr, ng2, W1, b1r, W2, b2r):
    return pl.pallas_call(
        _tc_mlp_kernel,
        grid=(_GRID,),
        in_specs=[
            pl.BlockSpec((_BLK, _D_FEAT), lambda i: (i, 0)),
            pl.BlockSpec((_NC, _NS, _BLK), lambda i: (0, 0, i)),
            pl.BlockSpec((_NC, _NS, _BLK), lambda i: (0, 0, i)),
            pl.BlockSpec((_N_BATCHES, _D_GLOBAL), lambda i: (0, 0)),
            pl.BlockSpec((1, _BLK), lambda i: (0, i)),
            pl.BlockSpec((_IN_DIM, _LATENT), lambda i: (0, 0)),
            pl.BlockSpec((1, _LATENT), lambda i: (0, 0)),
            pl.BlockSpec((_LATENT, _OUT_DIM), lambda i: (0, 0)),
            pl.BlockSpec((1, _OUT_DIM), lambda i: (0, 0)),
        ],
        out_specs=pl.BlockSpec((_BLK, _OUT_DIM), lambda i: (i, 0)),
        out_shape=jax.ShapeDtypeStruct((_N_NODES, _OUT_DIM), jnp.float32),
    )(node_attr, sums, cnts, global_attr, ng2, W1, b1r, W2, b2r)


def kernel(node_attr, edge_attr, global_attr, edge_index, ng_index, eg_index,
           W1, b1, W2, b2):
    eT = edge_attr.T
    zeros_n = jnp.zeros((_N_NODES,), jnp.float32)
    sums, cnts = _sc_scatter_mean_partials(eT, edge_index, zeros_n)
    ng2 = ng_index.reshape(1, _N_NODES)
    return _tc_mlp(node_attr, sums, cnts, global_attr, ng2,
                   W1, b1.reshape(1, -1), W2, b2.reshape(1, -1))
